# Initial kernel scaffold; baseline (speedup 1.0000x reference)
import jax
import jax.numpy as jnp
from jax.experimental import pallas as pl


def kernel(x, edge_index, W1, b1, W2, b2):
    raise NotImplementedError("write your pallas kernel here")


# speedup vs baseline: 12.0123x; 12.0123x over previous
import functools

import jax
import jax.numpy as jnp
from jax import lax
from jax.experimental import pallas as pl
from jax.experimental.pallas import tpu as pltpu
from jax.experimental.pallas import tpu_sc as plsc

N = 10000
NPAD = 10240
RPT = NPAD // 16
E = 320000
B = 128
TILES = 32
NB = -(-E // (TILES * B))
EPAD = TILES * NB * B
PAD = N
F32 = jnp.float32

R = 512
GRID = NPAD // R


def _mesh():
    return plsc.VectorSubcoreMesh(core_axis_name="c", subcore_axis_name="s")


def _deg_call(dsts):
    @functools.partial(
        pl.kernel,
        mesh=_mesh(),
        out_type=jax.ShapeDtypeStruct((2 * NPAD,), F32),
        scratch_types=[
            pltpu.VMEM((NB, B), jnp.int32),
            pltpu.VMEM((B,), F32),
            pltpu.VMEM((RPT,), F32),
            pltpu.VMEM_SHARED((NPAD,), F32),
        ],
    )
    def deg_kernel(dst_hbm, out_hbm, idx_v, ones_v, zrow_v, deg_sh):
        c = lax.axis_index("c")
        s = lax.axis_index("s")
        for i in range(B // 16):
            ones_v[pl.ds(16 * i, 16)] = jnp.full((16,), 1.0, F32)
        for i in range(RPT // 16):
            zrow_v[pl.ds(16 * i, 16)] = jnp.zeros((16,), F32)
        pltpu.sync_copy(zrow_v, deg_sh.at[pl.ds(s * RPT, RPT)])
        t = c * 16 + s
        pltpu.sync_copy(dst_hbm.at[t], idx_v)
        plsc.subcore_barrier()

        def body(j, carry):
            pltpu.sync_copy(ones_v, deg_sh.at[idx_v.at[j]], add=True)
            return carry

        lax.fori_loop(0, NB, body, 0)
        plsc.subcore_barrier()
        pltpu.sync_copy(deg_sh.at[pl.ds(s * RPT, RPT)],
                        out_hbm.at[pl.ds(c * NPAD + s * RPT, RPT)])

    return deg_kernel(dsts)


def _scatter_call(g, srcs, dsts, zeros, feat):
    @functools.partial(
        pl.kernel,
        mesh=_mesh(),
        out_type=jax.ShapeDtypeStruct((2 * NPAD, feat), F32),
        scratch_types=[
            pltpu.VMEM((NB, B), jnp.int32),
            pltpu.VMEM((NB, B), jnp.int32),
            pltpu.VMEM((B, feat), F32),
            pltpu.VMEM_SHARED((NPAD, feat), F32),
            pltpu.SemaphoreType.DMA,
        ],
    )
    def sc_kernel(g_hbm, srcs_hbm, dsts_hbm, z_hbm, out_hbm,
                  isrc_v, idst_v, rows_v, acc_sh, sem):
        c = lax.axis_index("c")
        s = lax.axis_index("s")
        pltpu.sync_copy(z_hbm.at[pl.ds(s * RPT, RPT)],
                        acc_sh.at[pl.ds(s * RPT, RPT)])
        t = c * 16 + s
        pltpu.sync_copy(srcs_hbm.at[t], isrc_v)
        pltpu.sync_copy(dsts_hbm.at[t], idst_v)
        plsc.subcore_barrier()

        def body(j, carry):
            pltpu.async_copy(g_hbm.at[isrc_v.at[j]], rows_v, sem).wait()
            pltpu.sync_copy(rows_v, acc_sh.at[idst_v.at[j]], add=True)
            return carry

        lax.fori_loop(0, NB, body, 0)
        plsc.subcore_barrier()
        pltpu.sync_copy(acc_sh.at[pl.ds(s * RPT, RPT)],
                        out_hbm.at[pl.ds(c * NPAD + s * RPT, RPT)])

    return sc_kernel(g, srcs, dsts, zeros)


def _tc1(xp, W1, d0, d1):
    def body(x_ref, w_ref, d0_ref, d1_ref, g_ref, dv_ref):
        deg = d0_ref[...] + d1_ref[...] + 1.0
        dinv = lax.rsqrt(deg)
        h = jnp.dot(x_ref[...], w_ref[...],
                    preferred_element_type=F32,
                    precision=lax.Precision.HIGHEST)
        g_ref[...] = h * dinv
        dv_ref[...] = jnp.broadcast_to(dinv, (R, 128))

    return pl.pallas_call(
        body,
        grid=(GRID,),
        in_specs=[
            pl.BlockSpec((R, 128), lambda i: (i, 0)),
            pl.BlockSpec((128, 128), lambda i: (0, 0)),
            pl.BlockSpec((R, 1), lambda i: (i, 0)),
            pl.BlockSpec((R, 1), lambda i: (i, 0)),
        ],
        out_specs=[
            pl.BlockSpec((R, 128), lambda i: (i, 0)),
            pl.BlockSpec((R, 128), lambda i: (i, 0)),
        ],
        out_shape=[
            jax.ShapeDtypeStruct((NPAD, 128), F32),
            jax.ShapeDtypeStruct((NPAD, 128), F32),
        ],
    )(xp, W1, d0, d1)


def _tc2(a0, a1, g1, dv, b1, W2):
    def body(a0_ref, a1_ref, g_ref, dv_ref, b_ref, w_ref, o_ref):
        dvb = dv_ref[...]
        pre = (a0_ref[...] + a1_ref[...] + g_ref[...]) * dvb + b_ref[...]
        h = jnp.maximum(pre, 0.0)
        h2 = jnp.dot(h, w_ref[...],
                     preferred_element_type=F32,
                     precision=lax.Precision.HIGHEST)
        o_ref[...] = jnp.concatenate(
            [h2 * dvb[:, :64], jnp.zeros((R, 64), F32)], axis=1)

    return pl.pallas_call(
        body,
        grid=(GRID,),
        in_specs=[
            pl.BlockSpec((R, 128), lambda i: (i, 0)),
            pl.BlockSpec((R, 128), lambda i: (i, 0)),
            pl.BlockSpec((R, 128), lambda i: (i, 0)),
            pl.BlockSpec((R, 128), lambda i: (i, 0)),
            pl.BlockSpec((1, 128), lambda i: (0, 0)),
            pl.BlockSpec((128, 64), lambda i: (0, 0)),
        ],
        out_specs=pl.BlockSpec((R, 128), lambda i: (i, 0)),
        out_shape=jax.ShapeDtypeStruct((NPAD, 128), F32),
    )(a0, a1, g1, dv, b1, W2)


def _tc3(a0, a1, g2, dv, b2):
    def body(a0_ref, a1_ref, g_ref, dv_ref, b_ref, o_ref):
        acc = a0_ref[...] + a1_ref[...] + g_ref[...]
        o_ref[...] = acc[:, :64] * dv_ref[...][:, :64] + b_ref[...]

    return pl.pallas_call(
        body,
        grid=(GRID,),
        in_specs=[
            pl.BlockSpec((R, 128), lambda i: (i, 0)),
            pl.BlockSpec((R, 128), lambda i: (i, 0)),
            pl.BlockSpec((R, 128), lambda i: (i, 0)),
            pl.BlockSpec((R, 128), lambda i: (i, 0)),
            pl.BlockSpec((1, 64), lambda i: (0, 0)),
        ],
        out_specs=pl.BlockSpec((R, 64), lambda i: (i, 0)),
        out_shape=jax.ShapeDtypeStruct((NPAD, 64), F32),
    )(a0, a1, g2, dv, b2)


def kernel(x, edge_index, W1, b1, W2, b2):
    ei = edge_index.astype(jnp.int32)
    padcol = jnp.full((2, EPAD - E), PAD, jnp.int32)
    eip = jnp.concatenate([ei, padcol], axis=1)
    srcs = eip[0].reshape(TILES, NB, B)
    dsts = eip[1].reshape(TILES, NB, B)
    xp = jnp.zeros((NPAD, 128), F32).at[:N].set(x)

    degp = _deg_call(dsts)
    d0 = degp[:NPAD, None]
    d1 = degp[NPAD:, None]

    g1, dv = _tc1(xp, W1, d0, d1)

    acc1 = _scatter_call(g1, srcs, dsts, jnp.zeros((NPAD, 128), F32), 128)
    g2 = _tc2(acc1[:NPAD], acc1[NPAD:], g1, dv, b1.reshape(1, 128), W2)

    acc2 = _scatter_call(g2, srcs, dsts, jnp.zeros((NPAD, 128), F32), 128)
    out = _tc3(acc2[:NPAD], acc2[NPAD:], g2, dv, b2.reshape(1, 64))
    return out[:N]
